# Initial kernel scaffold; baseline (speedup 1.0000x reference)
import jax
import jax.numpy as jnp
from jax.experimental import pallas as pl


def kernel(x, weight):
    raise NotImplementedError("write your pallas kernel here")


# speedup vs baseline: 1.9701x; 1.9701x over previous
import functools

import jax
import jax.numpy as jnp
from jax import lax
from jax.experimental import pallas as pl
from jax.experimental.pallas import tpu as pltpu
from jax.experimental.pallas import tpu_sc as plsc

N = 16384
V, D = 48, 3
LANES = 16
NUM_CORES = 2
NUM_SUBCORES = 16
NW = NUM_CORES * NUM_SUBCORES
CHUNK = N // NW
STEPS = CHUNK // LANES

_mesh = plsc.VectorSubcoreMesh(core_axis_name="c", subcore_axis_name="s")


@functools.partial(
    pl.kernel,
    mesh=_mesh,
    out_type=jax.ShapeDtypeStruct((N * D,), jnp.float32),
    compiler_params=pltpu.CompilerParams(needs_layout_passes=False),
    scratch_types=[
        pltpu.VMEM((CHUNK,), jnp.int32),
        pltpu.VMEM((V * D,), jnp.float32),
        pltpu.VMEM((CHUNK * D,), jnp.float32),
    ],
)
def _emb_lookup(x_hbm, w_hbm, out_hbm, idx_v, w_v, out_v):
    wid = lax.axis_index("s") * NUM_CORES + lax.axis_index("c")
    base = wid * CHUNK
    pltpu.sync_copy(w_hbm, w_v)
    pltpu.sync_copy(x_hbm.at[pl.ds(base, CHUNK)], idx_v)
    lane3 = lax.iota(jnp.int32, LANES) * D
    for j in range(STEPS):
        xv = idx_v[pl.ds(j * LANES, LANES)]
        fx = xv * D
        for c in range(D):
            vals = plsc.load_gather(w_v, [fx + c])
            plsc.store_scatter(out_v, [lane3 + (j * LANES * D + c)], vals)
    pltpu.sync_copy(out_v, out_hbm.at[pl.ds(base * D, CHUNK * D)])


def kernel(x, weight):
    flat = _emb_lookup(x.astype(jnp.int32), weight.reshape(-1))
    return flat.reshape(N, D)


# speedup vs baseline: 2.3341x; 1.1848x over previous
import functools

import jax
import jax.numpy as jnp
from jax import lax
from jax.experimental import pallas as pl
from jax.experimental.pallas import tpu as pltpu
from jax.experimental.pallas import tpu_sc as plsc

N = 16384
V, D = 48, 3
LANES = 16
NUM_CORES = 2
NUM_SUBCORES = 16
NW = NUM_CORES * NUM_SUBCORES
CHUNK = N // NW
STEPS = CHUNK // LANES

_mesh = plsc.VectorSubcoreMesh(core_axis_name="c", subcore_axis_name="s")


@functools.partial(
    pl.kernel,
    mesh=_mesh,
    out_type=jax.ShapeDtypeStruct((N, D), jnp.float32),
    compiler_params=pltpu.CompilerParams(needs_layout_passes=False),
    scratch_types=[
        pltpu.VMEM((CHUNK,), jnp.int32),
        pltpu.VMEM((V, D), jnp.float32),
        pltpu.VMEM((CHUNK, D), jnp.float32),
    ],
)
def _emb_lookup(x_hbm, w_hbm, out_hbm, idx_v, w_v, out_v):
    wid = lax.axis_index("s") * NUM_CORES + lax.axis_index("c")
    base = wid * CHUNK
    pltpu.sync_copy(w_hbm, w_v)
    pltpu.sync_copy(x_hbm.at[pl.ds(base, CHUNK)], idx_v)
    lane = lax.iota(jnp.int32, LANES)
    for j in range(STEPS):
        xv = idx_v[pl.ds(j * LANES, LANES)]
        rows = lane + (j * LANES)
        for c in range(D):
            col = jnp.full((LANES,), c, jnp.int32)
            vals = plsc.load_gather(w_v, [xv, col])
            plsc.store_scatter(out_v, [rows, col], vals)
    pltpu.sync_copy(out_v, out_hbm.at[pl.ds(base, CHUNK)])


def kernel(x, weight):
    return _emb_lookup(x, weight)


# speedup vs baseline: 2.4235x; 1.0383x over previous
import functools

import jax
import jax.numpy as jnp
from jax import lax
from jax.experimental import pallas as pl
from jax.experimental.pallas import tpu as pltpu
from jax.experimental.pallas import tpu_sc as plsc

N = 16384
V, D = 48, 3
LANES = 16
NUM_CORES = 2
NUM_SUBCORES = 16
NW = NUM_CORES * NUM_SUBCORES
CHUNK = N // NW
STEPS = CHUNK // LANES

_mesh = plsc.VectorSubcoreMesh(core_axis_name="c", subcore_axis_name="s")


@functools.partial(
    pl.kernel,
    mesh=_mesh,
    out_type=jax.ShapeDtypeStruct((N, D), jnp.float32),
    compiler_params=pltpu.CompilerParams(needs_layout_passes=False),
    scratch_types=[
        pltpu.VMEM((CHUNK,), jnp.int32),
        pltpu.VMEM((V * D,), jnp.float32),
        pltpu.VMEM((CHUNK, D), jnp.float32),
    ],
)
def _emb_lookup(x_hbm, w_hbm, out_hbm, idx_v, w_v, out_v):
    wid = lax.axis_index("s") * NUM_CORES + lax.axis_index("c")
    base = wid * CHUNK
    pltpu.sync_copy(w_hbm, w_v)
    pltpu.sync_copy(x_hbm.at[pl.ds(base, CHUNK)], idx_v)
    lane = lax.iota(jnp.int32, LANES)
    for j in range(STEPS):
        xv = idx_v[pl.ds(j * LANES, LANES)]
        fx = xv * D
        rows = lane + (j * LANES)
        for c in range(D):
            vals = plsc.load_gather(w_v, [fx + c])
            plsc.store_scatter(out_v, [rows, jnp.full((LANES,), c, jnp.int32)], vals)
    pltpu.sync_copy(out_v, out_hbm.at[pl.ds(base, CHUNK)])


def kernel(x, weight):
    return _emb_lookup(x, weight.reshape(-1))


# speedup vs baseline: 2.4795x; 1.0231x over previous
import functools

import jax
import jax.numpy as jnp
from jax import lax
from jax.experimental import pallas as pl
from jax.experimental.pallas import tpu as pltpu
from jax.experimental.pallas import tpu_sc as plsc

N = 16384
V, D = 48, 3
LANES = 16
NUM_CORES = 2
NUM_SUBCORES = 16
NW = NUM_CORES * NUM_SUBCORES
CHUNK = N // NW
STEPS = CHUNK // LANES

_mesh = plsc.VectorSubcoreMesh(core_axis_name="c", subcore_axis_name="s")


@functools.partial(
    pl.kernel,
    mesh=_mesh,
    out_type=jax.ShapeDtypeStruct((N, D), jnp.float32),
    compiler_params=pltpu.CompilerParams(needs_layout_passes=False),
    scratch_types=[
        pltpu.VMEM((CHUNK,), jnp.int32),
        pltpu.VMEM((V * D,), jnp.float32),
        pltpu.VMEM((CHUNK, D), jnp.float32),
        pltpu.SemaphoreType.DMA,
        pltpu.SemaphoreType.DMA,
    ],
)
def _emb_lookup(x_hbm, w_hbm, out_hbm, idx_v, w_v, out_v, sem_w, sem_x):
    wid = lax.axis_index("s") * NUM_CORES + lax.axis_index("c")
    base = wid * CHUNK
    cp_w = pltpu.async_copy(w_hbm, w_v, sem_w)
    cp_x = pltpu.async_copy(x_hbm.at[pl.ds(base, CHUNK)], idx_v, sem_x)
    cp_w.wait()
    cp_x.wait()
    lane = lax.iota(jnp.int32, LANES)
    for j in range(STEPS):
        xv = idx_v[pl.ds(j * LANES, LANES)]
        fx = xv * D
        rows = lane + (j * LANES)
        for c in range(D):
            vals = plsc.load_gather(w_v, [fx + c])
            plsc.store_scatter(out_v, [rows, jnp.full((LANES,), c, jnp.int32)], vals)
    pltpu.sync_copy(out_v, out_hbm.at[pl.ds(base, CHUNK)])


def kernel(x, weight):
    return _emb_lookup(x, weight.reshape(-1))


# speedup vs baseline: 2.5177x; 1.0154x over previous
import functools

import jax
import jax.numpy as jnp
from jax import lax
from jax.experimental import pallas as pl
from jax.experimental.pallas import tpu as pltpu
from jax.experimental.pallas import tpu_sc as plsc

N = 16384
V, D = 48, 3
LANES = 16
NUM_CORES = 2
NUM_SUBCORES = 16
NW = NUM_CORES * NUM_SUBCORES
CHUNK = N // NW
STEPS = CHUNK // LANES

_mesh = plsc.VectorSubcoreMesh(core_axis_name="c", subcore_axis_name="s")


@functools.partial(
    pl.kernel,
    mesh=_mesh,
    out_type=jax.ShapeDtypeStruct((N, D), jnp.float32),
    compiler_params=pltpu.CompilerParams(needs_layout_passes=False),
    scratch_types=[
        pltpu.VMEM((CHUNK,), jnp.int32),
        pltpu.VMEM((V * D,), jnp.float32),
        pltpu.VMEM((CHUNK, D), jnp.float32),
        pltpu.SemaphoreType.DMA,
        pltpu.SemaphoreType.DMA,
        pltpu.SemaphoreType.DMA,
        pltpu.SemaphoreType.DMA,
        pltpu.SemaphoreType.DMA,
    ],
)
def _emb_lookup(x_hbm, w_hbm, out_hbm, idx_v, w_v, out_v,
                sem_w, sem_x1, sem_x2, sem_o1, sem_o2):
    wid = lax.axis_index("s") * NUM_CORES + lax.axis_index("c")
    base = wid * CHUNK
    half = CHUNK // 2
    cp_w = pltpu.async_copy(w_hbm, w_v, sem_w)
    cp_x1 = pltpu.async_copy(x_hbm.at[pl.ds(base, half)],
                             idx_v.at[pl.ds(0, half)], sem_x1)
    cp_x2 = pltpu.async_copy(x_hbm.at[pl.ds(base + half, half)],
                             idx_v.at[pl.ds(half, half)], sem_x2)
    lane = lax.iota(jnp.int32, LANES)

    def steps(lo, hi):
        for j in range(lo, hi):
            xv = idx_v[pl.ds(j * LANES, LANES)]
            fx = xv * D
            rows = lane + (j * LANES)
            for c in range(D):
                vals = plsc.load_gather(w_v, [fx + c])
                plsc.store_scatter(
                    out_v, [rows, jnp.full((LANES,), c, jnp.int32)], vals)

    cp_w.wait()
    cp_x1.wait()
    steps(0, STEPS // 2)
    cp_o1 = pltpu.async_copy(out_v.at[pl.ds(0, half)],
                             out_hbm.at[pl.ds(base, half)], sem_o1)
    cp_x2.wait()
    steps(STEPS // 2, STEPS)
    cp_o2 = pltpu.async_copy(out_v.at[pl.ds(half, half)],
                             out_hbm.at[pl.ds(base + half, half)], sem_o2)
    cp_o1.wait()
    cp_o2.wait()


def kernel(x, weight):
    return _emb_lookup(x, weight.reshape(-1))


# speedup vs baseline: 2.5775x; 1.0237x over previous
import functools

import jax
import jax.numpy as jnp
from jax import lax
from jax.experimental import pallas as pl
from jax.experimental.pallas import tpu as pltpu
from jax.experimental.pallas import tpu_sc as plsc

N = 16384
V, D = 48, 3
LANES = 16
NUM_CORES = 2
NUM_SUBCORES = 16
NW = NUM_CORES * NUM_SUBCORES
CHUNK = N // NW
STEPS = CHUNK // LANES

_mesh = plsc.VectorSubcoreMesh(core_axis_name="c", subcore_axis_name="s")


@functools.partial(
    pl.kernel,
    mesh=_mesh,
    out_type=jax.ShapeDtypeStruct((N, D), jnp.float32),
    compiler_params=pltpu.CompilerParams(needs_layout_passes=False),
    scratch_types=[
        pltpu.VMEM((CHUNK,), jnp.int32),
        pltpu.VMEM((V * D,), jnp.float32),
        pltpu.VMEM((CHUNK, D), jnp.float32),
        pltpu.SemaphoreType.DMA,
        pltpu.SemaphoreType.DMA,
        pltpu.SemaphoreType.DMA,
        pltpu.SemaphoreType.DMA,
        pltpu.SemaphoreType.DMA,
    ],
)
def _emb_lookup(x_hbm, w_hbm, out_hbm, idx_v, w_v, out_v,
                sem_w, sem_x1, sem_x2, sem_o1, sem_o2):
    wid = lax.axis_index("s") * NUM_CORES + lax.axis_index("c")
    base = wid * CHUNK
    half = CHUNK // 2
    cp_w = pltpu.async_copy(w_hbm, w_v, sem_w)
    cp_x1 = pltpu.async_copy(x_hbm.at[pl.ds(base, half)],
                             idx_v.at[pl.ds(0, half)], sem_x1)
    cp_x2 = pltpu.async_copy(x_hbm.at[pl.ds(base + half, half)],
                             idx_v.at[pl.ds(half, half)], sem_x2)
    lane = lax.iota(jnp.int32, LANES)

    cols = [jnp.full((LANES,), c, jnp.int32) for c in range(D)]

    def steps(lo, hi):
        def body(j, _):
            xv = idx_v[pl.ds(j * LANES, LANES)]
            fx = xv * D
            rows = lane + (j * LANES)
            for c in range(D):
                vals = plsc.load_gather(w_v, [fx + c])
                plsc.store_scatter(out_v, [rows, cols[c]], vals)
            return 0
        lax.fori_loop(lo, hi, body, 0)

    cp_w.wait()
    cp_x1.wait()
    steps(0, STEPS // 2)
    cp_o1 = pltpu.async_copy(out_v.at[pl.ds(0, half)],
                             out_hbm.at[pl.ds(base, half)], sem_o1)
    cp_x2.wait()
    steps(STEPS // 2, STEPS)
    cp_o2 = pltpu.async_copy(out_v.at[pl.ds(half, half)],
                             out_hbm.at[pl.ds(base + half, half)], sem_o2)
    cp_o1.wait()
    cp_o2.wait()


def kernel(x, weight):
    return _emb_lookup(x, weight.reshape(-1))
